# Initial kernel scaffold; baseline (speedup 1.0000x reference)
import jax
import jax.numpy as jnp
from jax.experimental import pallas as pl


def kernel(pharm_h0, prot_h0, pharm_x0, prot_x0, timestep, pharm_batch_idx, prot_batch_idx, ff_edge_index, pf_src, pf_dst, pp_edge_index, params):
    raise NotImplementedError("write your pallas kernel here")


# speedup vs baseline: 8.1908x; 8.1908x over previous
import jax
import jax.numpy as jnp
from jax.experimental import pallas as pl

_S = 128
_V = 16


def _dot(a, b):
    return jax.lax.dot_general(a, b, (((1,), (0,)), ((), ())),
                               preferred_element_type=jnp.float32)


def _silu(x):
    return x * jax.nn.sigmoid(x)


def _gvp_block(feats, vecs, wh, wu, ws, bs, sigmoid_gate):
    vh = [_dot(v, wh) for v in vecs]
    sh = jnp.sqrt(vh[0] * vh[0] + vh[1] * vh[1] + vh[2] * vh[2] + 1e-8)
    s = jnp.concatenate([feats, sh], axis=1)
    fo = _silu(_dot(s, ws) + bs)
    vu = [_dot(h, wu) for h in vh]
    nrm = jnp.sqrt(vu[0] * vu[0] + vu[1] * vu[1] + vu[2] * vu[2] + 1e-8)
    gate = jax.nn.sigmoid(nrm) if sigmoid_gate else nrm
    return fo, [gate * u for u in vu]


def _chans(ref):
    return [ref[:, c * _V:(c + 1) * _V] for c in range(3)]


def _enc_body(h_ref, t_ref, w_ref, b_ref, g_ref, be_ref, o_ref):
    w = w_ref[...]
    z = _dot(h_ref[...], w[:-1, :]) + t_ref[...] * w[-1:, :] + b_ref[...]
    z = _silu(z)
    m = jnp.mean(z, axis=1, keepdims=True)
    v = jnp.mean((z - m) ** 2, axis=1, keepdims=True)
    o_ref[...] = (z - m) / jnp.sqrt(v + 1e-5) * g_ref[...] + be_ref[...]


def _full(a):
    return pl.BlockSpec(a.shape, lambda i: (0,) * a.ndim)


def _rows(bn, k):
    return pl.BlockSpec((bn, k), lambda i: (i, 0))


def _encode(h0, t, enc, bn):
    n, din = h0.shape
    w = enc['lin']['W']
    args = (h0, t, w, enc['lin']['b'].reshape(1, -1),
            enc['g'].reshape(1, -1), enc['b'].reshape(1, -1))
    return pl.pallas_call(
        _enc_body,
        grid=(pl.cdiv(n, bn),),
        in_specs=[_rows(bn, din), _rows(bn, 1)] + [_full(a) for a in args[2:]],
        out_specs=_rows(bn, _S),
        out_shape=jax.ShapeDtypeStruct((n, _S), jnp.float32),
    )(*args)


def _msg_body(ss_ref, sd_ref, xs_ref, xd_ref, vg_ref,
              wh1, wu1, ws1, bs1, wh2, wu2, ws2, bs2, wh3, wu3, ws3, bs3,
              os_ref, ov_ref):
    d = xd_ref[...] - xs_ref[...]
    dist = jnp.sqrt(jnp.sum(d * d, axis=1, keepdims=True) + 1e-8)
    dn = d / dist
    vecs = [jnp.concatenate([vg_ref[:, c * _V:(c + 1) * _V], dn[:, c:c + 1]],
                            axis=1) for c in range(3)]
    feats = jnp.concatenate([ss_ref[...], sd_ref[...], dist], axis=1)
    fo, vecs = _gvp_block(feats, vecs, wh1[...], wu1[...], ws1[...], bs1[...], True)
    fo, vecs = _gvp_block(fo, vecs, wh2[...], wu2[...], ws2[...], bs2[...], True)
    fo, vecs = _gvp_block(fo, vecs, wh3[...], wu3[...], ws3[...], bs3[...], True)
    os_ref[...] = fo
    ov_ref[...] = jnp.concatenate(vecs, axis=1)


def _messages(ss, sd, xs, xd, vg, stack, be):
    e = ss.shape[0]
    wargs = []
    for gp in stack:
        wargs += [gp['Wh'], gp['Wu'], gp['Ws'], gp['bs'].reshape(1, -1)]
    outs = pl.pallas_call(
        _msg_body,
        grid=(pl.cdiv(e, be),),
        in_specs=[_rows(be, _S), _rows(be, _S), _rows(be, 3), _rows(be, 3),
                  _rows(be, 3 * _V)] + [_full(a) for a in wargs],
        out_specs=[_rows(be, _S), _rows(be, 3 * _V)],
        out_shape=[jax.ShapeDtypeStruct((e, _S), jnp.float32),
                   jax.ShapeDtypeStruct((e, 3 * _V), jnp.float32)],
    )(ss, sd, xs, xd, vg, *wargs)
    return outs


def _upd_body(s_ref, as_ref, v_ref, av_ref,
              wh1, wu1, ws1, bs1, wh2, wu2, ws2, bs2, os_ref, ov_ref):
    s = s_ref[...]
    feats = jnp.concatenate([s, as_ref[...]], axis=1)
    vref = v_ref[...]
    aref = av_ref[...]
    vecs = [jnp.concatenate([vref[:, c * _V:(c + 1) * _V],
                             aref[:, c * _V:(c + 1) * _V]], axis=1)
            for c in range(3)]
    fo, vecs = _gvp_block(feats, vecs, wh1[...], wu1[...], ws1[...], bs1[...], True)
    fo, vecs = _gvp_block(fo, vecs, wh2[...], wu2[...], ws2[...], bs2[...], True)
    os_ref[...] = s + fo
    ov_ref[...] = vref + jnp.concatenate(vecs, axis=1)


def _update(s, ags, v, agv, stack, bn):
    n = s.shape[0]
    wargs = []
    for gp in stack:
        wargs += [gp['Wh'], gp['Wu'], gp['Ws'], gp['bs'].reshape(1, -1)]
    return pl.pallas_call(
        _upd_body,
        grid=(pl.cdiv(n, bn),),
        in_specs=[_rows(bn, _S), _rows(bn, _S), _rows(bn, 3 * _V),
                  _rows(bn, 3 * _V)] + [_full(a) for a in wargs],
        out_specs=[_rows(bn, _S), _rows(bn, 3 * _V)],
        out_shape=[jax.ShapeDtypeStruct((n, _S), jnp.float32),
                   jax.ShapeDtypeStruct((n, 3 * _V), jnp.float32)],
    )(s, ags, v, agv, *wargs)


def _noise_body(s_ref, v_ref, wh1, wu1, ws1, bs1, wh2, wu2, ws2, bs2,
                wh3, wu3, ws3, bs3, wo, bo, oh_ref, ox_ref):
    vecs = _chans(v_ref)
    fo, vecs = _gvp_block(s_ref[...], vecs, wh1[...], wu1[...], ws1[...], bs1[...], True)
    fo, vecs = _gvp_block(fo, vecs, wh2[...], wu2[...], ws2[...], bs2[...], True)
    fo, vecs = _gvp_block(fo, vecs, wh3[...], wu3[...], ws3[...], bs3[...], False)
    oh_ref[...] = _dot(fo, wo[...]) + bo[...]
    ox_ref[...] = jnp.concatenate(vecs, axis=1)


def _noise(s, v, gvps, out_lin, bn):
    n = s.shape[0]
    wargs = []
    for gp in gvps:
        wargs += [gp['Wh'], gp['Wu'], gp['Ws'], gp['bs'].reshape(1, -1)]
    wargs += [out_lin['W'], out_lin['b'].reshape(1, -1)]
    nh = out_lin['W'].shape[1]
    return pl.pallas_call(
        _noise_body,
        grid=(pl.cdiv(n, bn),),
        in_specs=[_rows(bn, _S), _rows(bn, 3 * _V)] + [_full(a) for a in wargs],
        out_specs=[_rows(bn, nh), _rows(bn, 3)],
        out_shape=[jax.ShapeDtypeStruct((n, nh), jnp.float32),
                   jax.ShapeDtypeStruct((n, 3), jnp.float32)],
    )(s, v, *wargs)


def kernel(pharm_h0, prot_h0, pharm_x0, prot_x0, timestep, pharm_batch_idx,
           prot_batch_idx, ff_edge_index, pf_src, pf_dst, pp_edge_index,
           params):
    np_, nr_ = pharm_h0.shape[0], prot_h0.shape[0]
    bn = 2000
    be = 1000

    s_ph = _encode(pharm_h0, timestep[pharm_batch_idx][:, None],
                   params['pharm_enc'], bn)
    s_pr = _encode(prot_h0, timestep[prot_batch_idx][:, None],
                   params['prot_enc'], bn)

    node = {'pharm': [s_ph, pharm_x0, jnp.zeros((np_, 3 * _V), jnp.float32)],
            'prot': [s_pr, prot_x0, jnp.zeros((nr_, 3 * _V), jnp.float32)]}
    nnode = {'pharm': np_, 'prot': nr_}
    edges = {'ff': ('pharm', 'pharm', ff_edge_index[0], ff_edge_index[1]),
             'pf': ('prot', 'pharm', pf_src, pf_dst),
             'fp': ('pharm', 'prot', pf_dst, pf_src),
             'pp': ('prot', 'prot', pp_edge_index[0], pp_edge_index[1])}

    for conv in params['convs']:
        agg = {nt: [jnp.zeros((nnode[nt], _S), jnp.float32),
                    jnp.zeros((nnode[nt], 3 * _V), jnp.float32)]
               for nt in node}
        for et in ['ff', 'pf', 'fp', 'pp']:
            st, dt, src, dst = edges[et]
            s_s, x_s, v_s = node[st]
            s_d, x_d, _ = node[dt]
            msg_s, msg_v = _messages(s_s[src], s_d[dst], x_s[src], x_d[dst],
                                     v_s[src], conv['msg'][et], be)
            agg[dt][0] = agg[dt][0] + jax.ops.segment_sum(
                msg_s, dst, num_segments=nnode[dt])
            agg[dt][1] = agg[dt][1] + jax.ops.segment_sum(
                msg_v, dst, num_segments=nnode[dt])
        for nt in ['pharm', 'prot']:
            s, x, v = node[nt]
            s_new, v_new = _update(s, agg[nt][0], v, agg[nt][1],
                                   conv['upd'][nt], bn)
            node[nt] = [s_new, x, v_new]

    s, x, v = node['pharm']
    eps_h, eps_x = _noise(s, v, params['noise']['gvps'],
                          params['noise']['out'], bn)
    return (eps_h, eps_x)


# speedup vs baseline: 15.3281x; 1.8714x over previous
import functools

import jax
import jax.numpy as jnp
from jax import lax
from jax.experimental import pallas as pl
from jax.experimental.pallas import tpu as pltpu
from jax.experimental.pallas import tpu_sc as plsc

_S = 128
_V = 16
_TD = 256
_B = 64
_W = 256


def _dot(a, b):
    return jax.lax.dot_general(a, b, (((1,), (0,)), ((), ())),
                               preferred_element_type=jnp.float32)


def _silu(x):
    return x * jax.nn.sigmoid(x)


def _gvp_block(feats, vecs, wh, wu, ws, bs, sigmoid_gate):
    vh = [_dot(v, wh) for v in vecs]
    sh = jnp.sqrt(vh[0] * vh[0] + vh[1] * vh[1] + vh[2] * vh[2] + 1e-8)
    s = jnp.concatenate([feats, sh], axis=1)
    fo = _silu(_dot(s, ws) + bs)
    vu = [_dot(h, wu) for h in vh]
    nrm = jnp.sqrt(vu[0] * vu[0] + vu[1] * vu[1] + vu[2] * vu[2] + 1e-8)
    gate = jax.nn.sigmoid(nrm) if sigmoid_gate else nrm
    return fo, [gate * u for u in vu]


def _full(a):
    return pl.BlockSpec(a.shape, lambda i: (0,) * a.ndim)


def _rows(bn, k):
    return pl.BlockSpec((bn, k), lambda i: (i, 0))


def _wstack(stack):
    wargs = []
    for gp in stack:
        wargs += [gp['Wh'], gp['Wu'], gp['Ws'], gp['bs'].reshape(1, -1)]
    return wargs


def _enc_body(h_ref, bi_ref, x_ref, ts_ref, w_ref, b_ref, g_ref, be_ref,
              o_ref):
    bn = h_ref.shape[0]
    oh = (bi_ref[...] == lax.broadcasted_iota(jnp.int32, (bn, _B), 1))
    t = _dot(oh.astype(jnp.float32), ts_ref[...])
    w = w_ref[...]
    z = _dot(h_ref[...], w[:-1, :]) + t * w[-1:, :] + b_ref[...]
    z = _silu(z)
    m = jnp.mean(z, axis=1, keepdims=True)
    v = jnp.mean((z - m) ** 2, axis=1, keepdims=True)
    s = (z - m) / jnp.sqrt(v + 1e-5) * g_ref[...] + be_ref[...]
    o_ref[...] = jnp.concatenate(
        [s, jnp.zeros((bn, 3 * _V), jnp.float32), x_ref[...],
         jnp.zeros((bn, _TD - _S - 3 * _V - 3), jnp.float32)], axis=1)


def _encode(h0, bidx, x, timestep, enc, bn):
    n, din = h0.shape
    args = (h0, bidx.reshape(-1, 1).astype(jnp.int32), x,
            timestep.reshape(_B, 1), enc['lin']['W'],
            enc['lin']['b'].reshape(1, -1), enc['g'].reshape(1, -1),
            enc['b'].reshape(1, -1))
    return pl.pallas_call(
        _enc_body,
        grid=(pl.cdiv(n, bn),),
        in_specs=[_rows(bn, din), _rows(bn, 1), _rows(bn, 3)] +
                 [_full(a) for a in args[3:]],
        out_specs=_rows(bn, _TD),
        out_shape=jax.ShapeDtypeStruct((n, _TD), jnp.float32),
    )(*args)


def _make_gather(e_pad):
    epw = e_pad // 32
    nfull, tail = divmod(epw, _W)
    mesh = plsc.VectorSubcoreMesh(core_axis_name="c", subcore_axis_name="s")

    def body(ts, td, is_, id_, os_, od_, idx_v, rows_v, sem):
        wid = lax.axis_index("s") * 2 + lax.axis_index("c")
        base = wid * epw

        def one(off, w, tab, ih, oh):
            if w == _W:
                pltpu.sync_copy(ih.at[pl.ds(off, _W)], idx_v)
            else:
                pltpu.sync_copy(ih.at[pl.ds(off, w)], idx_v.at[pl.ds(0, w)])
            pltpu.async_copy(tab.at[idx_v], rows_v, sem).wait()
            pltpu.sync_copy(rows_v.at[pl.ds(0, w)], oh.at[pl.ds(off, w)])

        def win(g, carry):
            off = base + g * _W
            one(off, _W, ts, is_, os_)
            one(off, _W, td, id_, od_)
            return carry

        lax.fori_loop(0, nfull, win, 0)
        if tail:
            off = base + nfull * _W
            one(off, tail, ts, is_, os_)
            one(off, tail, td, id_, od_)

    kern = functools.partial(
        pl.kernel, mesh=mesh,
        out_type=[jax.ShapeDtypeStruct((e_pad, _TD), jnp.float32)] * 2,
        scratch_types=[pltpu.VMEM((_W,), jnp.int32),
                       pltpu.VMEM((_W, _TD), jnp.float32),
                       pltpu.SemaphoreType.DMA],
    )(body)
    return kern


_DC = _TD // 2


def _make_scatter(nb, rr, tr, zc, dc_, e_pads):
    rpb = rr + tr
    mesh = plsc.VectorSubcoreMesh(core_axis_name="c", subcore_axis_name="s")
    ne = len(e_pads)

    def body(*refs):
        msgs = refs[0:ne * (nb + 1):nb + 1]
        idxs = [refs[k * (nb + 1) + 1:(k + 1) * (nb + 1)] for k in range(ne)]
        out = refs[ne * (nb + 1)]
        idx_v, msg_v, zb, acc = refs[ne * (nb + 1) + 1:]
        cid = lax.axis_index("c")
        sid = lax.axis_index("s")
        c0 = cid * _DC
        row0 = sid * rpb

        def zrow(r, carry):
            for l in range(_DC // 16):
                zb[r, pl.ds(l * 16, 16)] = jnp.zeros((16,), jnp.float32)
            return carry
        lax.fori_loop(0, zc, zrow, 0)

        for b in range(nb):
            def zcopy(z, carry):
                pltpu.sync_copy(zb, acc.at[pl.ds(row0 + z * zc, zc)])
                return carry
            lax.fori_loop(0, rpb // zc, zcopy, 0)
            plsc.subcore_barrier()

            for k, e_pad in enumerate(e_pads):
                mh = msgs[k]
                ih = idxs[k][b]
                eps = e_pad // 16
                assert eps % _W == 0
                base = sid * eps

                def wloop(g, carry):
                    off = base + g * _W
                    pltpu.sync_copy(ih.at[pl.ds(off, _W)], idx_v)
                    pltpu.sync_copy(mh.at[pl.ds(off, _W), pl.ds(c0, _DC)],
                                    msg_v)
                    pltpu.sync_copy(msg_v, acc.at[idx_v], add=True)
                    return carry
                lax.fori_loop(0, eps // _W, wloop, 0)

            plsc.subcore_barrier()

            def dump(z, carry):
                o = b * 16 * rr + sid * rr + z * dc_
                pltpu.sync_copy(acc.at[pl.ds(row0 + z * dc_, dc_)],
                                out.at[pl.ds(o, dc_), pl.ds(c0, _DC)])
                return carry
            lax.fori_loop(0, rr // dc_, dump, 0)
            plsc.subcore_barrier()

    kern = functools.partial(
        pl.kernel, mesh=mesh,
        out_type=jax.ShapeDtypeStruct((nb * 16 * rr, _TD), jnp.float32),
        scratch_types=[pltpu.VMEM((_W,), jnp.int32),
                       pltpu.VMEM((_W, _DC), jnp.float32),
                       pltpu.VMEM((zc, _DC), jnp.float32),
                       pltpu.VMEM_SHARED((16 * rpb, _DC), jnp.float32)],
    )(body)
    return kern


def _remap_blocks(idx, nb, rr, tr):
    if nb == 1 and tr == 0:
        return [idx]
    rpb = rr + tr
    out = []
    trash = (idx % 16) * rpb + rr + ((idx // 16) % tr)
    for b in range(nb):
        rel = idx - b * (16 * rr)
        inb = (rel >= 0) & (rel < 16 * rr)
        srel = jnp.where(inb, rel, 0)
        out.append(jnp.where(inb, (srel // rr) * rpb + srel % rr,
                             trash).astype(jnp.int32))
    return out


def _msg_body(e_real, gs_ref, gd_ref,
              wh1, wu1, ws1, bs1, wh2, wu2, ws2, bs2, wh3, wu3, ws3, bs3,
              o_ref):
    be = gs_ref.shape[0]
    gs = gs_ref[...]
    gd = gd_ref[...]
    xs = gs[:, _S + 3 * _V:_S + 3 * _V + 3]
    xd = gd[:, _S + 3 * _V:_S + 3 * _V + 3]
    d = xd - xs
    dist = jnp.sqrt(jnp.sum(d * d, axis=1, keepdims=True) + 1e-8)
    dn = d / dist
    vecs = [jnp.concatenate([gs[:, _S + c * _V:_S + (c + 1) * _V],
                             dn[:, c:c + 1]], axis=1) for c in range(3)]
    feats = jnp.concatenate([gs[:, :_S], gd[:, :_S], dist], axis=1)
    fo, vecs = _gvp_block(feats, vecs, wh1[...], wu1[...], ws1[...], bs1[...], True)
    fo, vecs = _gvp_block(fo, vecs, wh2[...], wu2[...], ws2[...], bs2[...], True)
    fo, vecs = _gvp_block(fo, vecs, wh3[...], wu3[...], ws3[...], bs3[...], True)
    rid = pl.program_id(0) * be + lax.broadcasted_iota(jnp.int32, (be, 1), 0)
    live = (rid < e_real).astype(jnp.float32)
    out = jnp.concatenate(
        [fo, jnp.concatenate(vecs, axis=1),
         jnp.zeros((be, _TD - _S - 3 * _V), jnp.float32)], axis=1)
    o_ref[...] = out * live


def _messages(gs, gd, e_real, stack, be):
    e = gs.shape[0]
    wargs = _wstack(stack)
    return pl.pallas_call(
        functools.partial(_msg_body, e_real),
        grid=(pl.cdiv(e, be),),
        in_specs=[_rows(be, _TD), _rows(be, _TD)] + [_full(a) for a in wargs],
        out_specs=_rows(be, _TD),
        out_shape=jax.ShapeDtypeStruct((e, _TD), jnp.float32),
    )(gs, gd, *wargs)


def _upd_body(t_ref, a_ref,
              wh1, wu1, ws1, bs1, wh2, wu2, ws2, bs2, o_ref):
    bn = t_ref.shape[0]
    tab = t_ref[...]
    agg = a_ref[...]
    s = tab[:, :_S]
    feats = jnp.concatenate([s, agg[:, :_S]], axis=1)
    vecs = [jnp.concatenate([tab[:, _S + c * _V:_S + (c + 1) * _V],
                             agg[:, _S + c * _V:_S + (c + 1) * _V]], axis=1)
            for c in range(3)]
    fo, vecs = _gvp_block(feats, vecs, wh1[...], wu1[...], ws1[...], bs1[...], True)
    fo, vecs = _gvp_block(fo, vecs, wh2[...], wu2[...], ws2[...], bs2[...], True)
    vnew = tab[:, _S:_S + 3 * _V] + jnp.concatenate(vecs, axis=1)
    o_ref[...] = jnp.concatenate(
        [s + fo, vnew, tab[:, _S + 3 * _V:_S + 3 * _V + 3],
         jnp.zeros((bn, _TD - _S - 3 * _V - 3), jnp.float32)], axis=1)


def _update(tab, agg, stack, bn):
    n = tab.shape[0]
    wargs = _wstack(stack)
    return pl.pallas_call(
        _upd_body,
        grid=(pl.cdiv(n, bn),),
        in_specs=[_rows(bn, _TD), _rows(bn, _TD)] + [_full(a) for a in wargs],
        out_specs=_rows(bn, _TD),
        out_shape=jax.ShapeDtypeStruct((n, _TD), jnp.float32),
    )(tab, agg, *wargs)


def _noise_body(t_ref, wh1, wu1, ws1, bs1, wh2, wu2, ws2, bs2,
                wh3, wu3, ws3, bs3, wo, bo, oh_ref, ox_ref):
    tab = t_ref[...]
    vecs = [tab[:, _S + c * _V:_S + (c + 1) * _V] for c in range(3)]
    fo, vecs = _gvp_block(tab[:, :_S], vecs, wh1[...], wu1[...], ws1[...], bs1[...], True)
    fo, vecs = _gvp_block(fo, vecs, wh2[...], wu2[...], ws2[...], bs2[...], True)
    fo, vecs = _gvp_block(fo, vecs, wh3[...], wu3[...], ws3[...], bs3[...], False)
    oh_ref[...] = _dot(fo, wo[...]) + bo[...]
    ox_ref[...] = jnp.concatenate(vecs, axis=1)


def _noise(tab, gvps, out_lin, bn):
    n = tab.shape[0]
    wargs = _wstack(gvps) + [out_lin['W'], out_lin['b'].reshape(1, -1)]
    nh = out_lin['W'].shape[1]
    return pl.pallas_call(
        _noise_body,
        grid=(pl.cdiv(n, bn),),
        in_specs=[_rows(bn, _TD)] + [_full(a) for a in wargs],
        out_specs=[_rows(bn, nh), _rows(bn, 3)],
        out_shape=[jax.ShapeDtypeStruct((n, nh), jnp.float32),
                   jax.ShapeDtypeStruct((n, 3), jnp.float32)],
    )(tab, *wargs)


def _pad_idx(idx, e_pad, n):
    e = idx.shape[0]
    if e == e_pad:
        return idx.astype(jnp.int32)
    fill = jnp.arange(e_pad - e, dtype=jnp.int32) % n
    return jnp.concatenate([idx.astype(jnp.int32), fill])


def kernel(pharm_h0, prot_h0, pharm_x0, prot_x0, timestep, pharm_batch_idx,
           prot_batch_idx, ff_edge_index, pf_src, pf_dst, pp_edge_index,
           params):
    np_, nr_ = pharm_h0.shape[0], prot_h0.shape[0]
    e_ff = ff_edge_index.shape[1]
    e_pf = pf_src.shape[0]
    e_pp = pp_edge_index.shape[1]
    bn = 2000
    be = 1024

    pad = lambda e: ((e + 4095) // 4096) * 4096
    ep_ff, ep_pf, ep_pp = pad(e_ff), pad(e_pf), pad(e_pp)
    ffs = _pad_idx(ff_edge_index[0], ep_ff, np_)
    ffd = _pad_idx(ff_edge_index[1], ep_ff, np_)
    pfs = _pad_idx(pf_src, ep_pf, nr_)
    pfd = _pad_idx(pf_dst, ep_pf, np_)
    pps = _pad_idx(pp_edge_index[0], ep_pp, nr_)
    ppd = _pad_idx(pp_edge_index[1], ep_pp, nr_)

    gather_ff = _make_gather(ep_ff)
    gather_pf = _make_gather(ep_pf)
    gather_pp = _make_gather(ep_pp)
    scat_pharm = _make_scatter(1, 640, 0, 64, 64, [ep_ff, ep_pf])
    scat_prot = _make_scatter(5, 640, 16, 16, 64, [ep_pf, ep_pp])
    ffd_b = _remap_blocks(ffd, 1, 640, 0)
    pfd_b = _remap_blocks(pfd, 1, 640, 0)
    pfs_b = _remap_blocks(pfs, 5, 640, 16)
    ppd_b = _remap_blocks(ppd, 5, 640, 16)

    tp = _encode(pharm_h0, pharm_batch_idx, pharm_x0, timestep,
                 params['pharm_enc'], bn)
    tr = _encode(prot_h0, prot_batch_idx, prot_x0, timestep,
                 params['prot_enc'], bn)

    for conv in params['convs']:
        g_ffs, g_ffd = gather_ff(tp, tp, ffs, ffd)
        g_pfs, g_pfd = gather_pf(tr, tp, pfs, pfd)
        g_pps, g_ppd = gather_pp(tr, tr, pps, ppd)
        m_ff = _messages(g_ffs, g_ffd, e_ff, conv['msg']['ff'], be)
        m_pf = _messages(g_pfs, g_pfd, e_pf, conv['msg']['pf'], be)
        m_fp = _messages(g_pfd, g_pfs, e_pf, conv['msg']['fp'], be)
        m_pp = _messages(g_pps, g_ppd, e_pp, conv['msg']['pp'], be)
        agg_p = scat_pharm(m_ff, *ffd_b, m_pf, *pfd_b)[:np_]
        agg_r = scat_prot(m_fp, *pfs_b, m_pp, *ppd_b)[:nr_]
        tp = _update(tp, agg_p, conv['upd']['pharm'], bn)
        tr = _update(tr, agg_r, conv['upd']['prot'], bn)

    eps_h, eps_x = _noise(tp, params['noise']['gvps'],
                          params['noise']['out'], bn)
    return (eps_h, eps_x)
